# baseline (speedup 1.0000x reference)
import functools

import jax
import jax.numpy as jnp
from jax import lax
from jax.experimental import pallas as pl
from jax.experimental.pallas import tpu as pltpu
from jax.experimental.pallas import tpu_sc as plsc


_LG = 15
_CH = 1 << _LG
_NCHUNK = 31
_NTC = 17
_RND = 0x8000
_MSK = -65536


def _pack_word(a_f32, b_f32):
    a = lax.bitcast_convert_type(a_f32, jnp.int32)
    b = lax.bitcast_convert_type(b_f32, jnp.int32)
    w = lax.bitwise_or(
        lax.bitwise_and(a + _RND, _MSK),
        lax.shift_right_logical(b + _RND, 16),
    )
    return lax.bitcast_convert_type(w, jnp.float32)


def _pack_body(xt_ref, o_ref):
    xt = xt_ref[...]
    d = xt.shape[0]
    eye = (
        lax.broadcasted_iota(jnp.int32, (d, d), 0)
        == lax.broadcasted_iota(jnp.int32, (d, d), 1)
    ).astype(jnp.float32)
    t = lax.dot_general(
        xt, eye, (((0,), (0,)), ((), ())),
        preferred_element_type=jnp.float32,
    )
    ch = t.shape[0]
    w1 = _pack_word(t[: ch // 2], t[ch // 2:])
    q = ch // 4
    o_ref[:, : w1.shape[1]] = w1[:q]
    o_ref[:, w1.shape[1]:] = w1[q:]


def _pack_tc(table_t):
    D, V = table_t.shape
    last = _NCHUNK - 1

    def _cb(i):
        return jnp.where(i < _NTC, i, last)

    return pl.pallas_call(
        _pack_body,
        grid=(_NTC + 1,),
        in_specs=[pl.BlockSpec((D, _CH), lambda i: (0, _cb(i)))],
        out_specs=pl.BlockSpec((_CH // 4, 2 * D), lambda i: (_cb(i), 0)),
        out_shape=jax.ShapeDtypeStruct((_NCHUNK * (_CH // 4), 2 * D),
                                       jnp.float32),
        compiler_params=pltpu.CompilerParams(fuse_transposed_lhs_in_matmul=True),
    )(table_t)


def _make_pack_sc(D, V):
    info = plsc.get_sparse_core_info()
    NC, NS = info.num_cores, info.num_subcores
    NW = NC * NS
    nsc = _NCHUNK - 1 - _NTC
    ngroups = nsc * 64
    per_w = ngroups // NW
    mesh = plsc.VectorSubcoreMesh(core_axis_name="c", subcore_axis_name="s")

    @functools.partial(
        pl.kernel,
        mesh=mesh,
        out_type=jax.ShapeDtypeStruct((_NCHUNK * (_CH // 4), 2 * D),
                                      jnp.float32),
        scratch_types=[
            pltpu.VMEM((2, 4, D, 128), jnp.float32),
            pltpu.VMEM((2, 128, 2 * D), jnp.float32),
            pltpu.SemaphoreType.DMA,
            pltpu.SemaphoreType.DMA,
            pltpu.SemaphoreType.DMA,
        ],
        compiler_params=pltpu.CompilerParams(needs_layout_passes=False),
    )
    def pack_sc_k(table_t_hbm, out_hbm, slabs_v, packed_v, s0, s1, so):
        wid = lax.axis_index("s") * NC + lax.axis_index("c")
        g0 = wid * per_w
        sems = (s0, s1)

        def fire(gi, buf):
            g = g0 + gi
            c = _NTC + g // 64
            tp = g % 64
            for q in range(4):
                col = (c * 256 + q * 64 + tp) * 128
                pltpu.async_copy(
                    table_t_hbm.at[:, pl.ds(col, 128)],
                    slabs_v.at[buf, q],
                    sems[buf],
                )

        fire(0, 0)
        for gi in range(per_w):
            buf = gi & 1
            if gi + 1 < per_w:
                fire(gi + 1, 1 - buf)
            for q in range(4):
                pltpu.make_async_copy(
                    table_t_hbm.at[:, pl.ds(0, 128)],
                    slabs_v.at[buf, q],
                    sems[buf],
                ).wait()
            if gi >= 2:
                pltpu.make_async_copy(
                    out_hbm.at[pl.ds(0, 128)],
                    packed_v.at[buf],
                    so,
                ).wait()

            def body(f, _):
                for half in range(2):
                    qa, qb = (0, 2) if half == 0 else (1, 3)
                    for jb in range(8):
                        sl = pl.ds(jb * 16, 16)
                        a = slabs_v[buf, qa, f, sl]
                        b = slabs_v[buf, qb, f, sl]
                        w = _pack_word(a, b)
                        rows = jb * 16 + lax.iota(jnp.int32, 16)
                        cols = jnp.full((16,), half * D, jnp.int32) + f
                        plsc.store_scatter(
                            packed_v.at[buf], [rows, cols], w
                        )
                return 0

            lax.fori_loop(0, D, body, 0)

            g = g0 + gi
            c = _NTC + g // 64
            tp = g % 64
            j0 = c * (_CH // 4) + tp * 128
            pltpu.async_copy(
                packed_v.at[buf], out_hbm.at[pl.ds(j0, 128)], so
            )
        for _ in range(2 if per_w >= 2 else per_w):
            pltpu.make_async_copy(
                out_hbm.at[pl.ds(0, 128)],
                packed_v.at[0],
                so,
            ).wait()

    return pack_sc_k


def _make_gather(D2, B):
    info = plsc.get_sparse_core_info()
    NC, NS = info.num_cores, info.num_subcores
    NW = NC * NS
    b_per_w = B // NW
    mesh = plsc.VectorSubcoreMesh(core_axis_name="c", subcore_axis_name="s")

    @functools.partial(
        pl.kernel,
        mesh=mesh,
        out_type=(
            jax.ShapeDtypeStruct((B, D2), jnp.float32),
            jax.ShapeDtypeStruct((B, D2), jnp.float32),
        ),
        scratch_types=[
            pltpu.VMEM((b_per_w,), jnp.int32),
            pltpu.VMEM((b_per_w,), jnp.int32),
            pltpu.VMEM((b_per_w, D2), jnp.float32),
            pltpu.SemaphoreType.DMA,
        ],
    )
    def gather_k(idx_hbm, ptc_hbm, psc_hbm, otc_hbm, osc_hbm,
                 idx_v, idx2_v, rows_v, sem):
        wid = lax.axis_index("s") * NC + lax.axis_index("c")
        base = wid * b_per_w
        pltpu.sync_copy(idx_hbm.at[pl.ds(base, b_per_w)], idx_v)
        for g in range(b_per_w // 16):
            sl = pl.ds(g * 16, 16)
            iv = idx_v[sl]
            idx2_v[sl] = lax.bitwise_or(
                lax.shift_left(lax.shift_right_logical(iv, _LG), _LG - 2),
                lax.bitwise_and(iv, (1 << (_LG - 2)) - 1),
            )
        pltpu.async_copy(ptc_hbm.at[idx2_v], rows_v, sem).wait()
        pltpu.sync_copy(rows_v, otc_hbm.at[pl.ds(base, b_per_w)])
        pltpu.async_copy(psc_hbm.at[idx2_v], rows_v, sem).wait()
        pltpu.sync_copy(rows_v, osc_hbm.at[pl.ds(base, b_per_w)])

    return gather_k


def _mm_body(xtc_ref, xsc_ref, s_ref, wt_ref, b_ref, o_ref):
    d = wt_ref.shape[0]
    s = s_ref[...]
    t_tc = lax.transpose(xtc_ref[...], (1, 0))
    t_sc = lax.transpose(xsc_ref[...], (1, 0))
    scbit = lax.bitwise_and(s, 4) == 4
    xt = jnp.where(scbit, t_sc, t_tc)
    colhalf = lax.bitwise_and(s, 1) == 1
    lohalf = lax.bitwise_and(s, 2) == 2
    half = jnp.where(colhalf, xt[d:, :], xt[:d, :])
    bits = lax.bitcast_convert_type(half, jnp.int32)
    bits = jnp.where(
        lohalf,
        lax.shift_left(bits, 16),
        lax.bitwise_and(bits, _MSK),
    )
    xsel = lax.bitcast_convert_type(bits, jnp.float32)
    o_ref[...] = (
        jnp.dot(wt_ref[...], xsel, preferred_element_type=jnp.float32)
        + b_ref[...]
    )


def _unpack_matmul_t(rows_tc, rows_sc, slot, wt, b2d):
    B, D2 = rows_tc.shape
    D = D2 // 2
    blk = 2048
    return pl.pallas_call(
        _mm_body,
        grid=(B // blk,),
        in_specs=[
            pl.BlockSpec((blk, D2), lambda i: (i, 0)),
            pl.BlockSpec((blk, D2), lambda i: (i, 0)),
            pl.BlockSpec((1, blk), lambda i: (0, i)),
            pl.BlockSpec((D, D), lambda i: (0, 0)),
            pl.BlockSpec((D, 1), lambda i: (0, 0)),
        ],
        out_specs=pl.BlockSpec((D, blk), lambda i: (0, i)),
        out_shape=jax.ShapeDtypeStruct((D, B), jnp.float32),
    )(rows_tc, rows_sc, slot, wt, b2d)


def kernel(fixed_features, fixed_table, W, b):
    V, D = fixed_table.shape
    B = fixed_features.shape[0]
    table_t = fixed_table.T
    packed_tc = _pack_tc(table_t)
    packed_sc = _make_pack_sc(D, V)(table_t)
    rows_tc, rows_sc = _make_gather(2 * D, B)(
        fixed_features, packed_tc, packed_sc
    )
    chunk = fixed_features >> _LG
    scbit = ((chunk >= _NTC) & (chunk < _NCHUNK - 1)).astype(jnp.int32)
    slot = (((fixed_features >> (_LG - 2)) & 3) | (scbit << 2)).reshape(1, B)
    wtop_t = W.T[:, :D]
    out_t = _unpack_matmul_t(rows_tc, rows_sc, slot, wtop_t, b.reshape(D, 1))
    return out_t.T


# speedup vs baseline: 1.7884x; 1.7884x over previous
import functools

import jax
import jax.numpy as jnp
from jax import lax
from jax.experimental import pallas as pl
from jax.experimental.pallas import tpu as pltpu
from jax.experimental.pallas import tpu_sc as plsc


_LG = 16


def _pack_body(xt_ref, o_ref):
    xt = xt_ref[...]
    d = xt.shape[0]
    eye = (
        lax.broadcasted_iota(jnp.int32, (d, d), 0)
        == lax.broadcasted_iota(jnp.int32, (d, d), 1)
    ).astype(jnp.float32)
    t = lax.dot_general(
        xt, eye, (((0,), (0,)), ((), ())),
        preferred_element_type=jnp.float32,
    )
    ch = t.shape[0]
    a = lax.bitcast_convert_type(t[: ch // 2], jnp.int32)
    b_ = lax.bitcast_convert_type(t[ch // 2:], jnp.int32)
    hi = lax.bitwise_and(a + 0x8000, jnp.int32(-65536))
    lo = lax.shift_right_logical(b_ + 0x8000, 16)
    w1 = lax.bitcast_convert_type(lax.bitwise_or(hi, lo), jnp.float32)
    q = ch // 4
    o_ref[:, : w1.shape[1]] = w1[:q]
    o_ref[:, w1.shape[1]:] = w1[q:]


def _pack(table_t):
    D, V = table_t.shape
    ch = 1 << _LG
    grid = (V + ch - 1) // ch
    return pl.pallas_call(
        _pack_body,
        grid=(grid,),
        in_specs=[pl.BlockSpec((D, ch), lambda i: (0, i))],
        out_specs=pl.BlockSpec((ch // 4, 2 * D), lambda i: (i, 0)),
        out_shape=jax.ShapeDtypeStruct((grid * (ch // 4), 2 * D), jnp.float32),
        compiler_params=pltpu.CompilerParams(fuse_transposed_lhs_in_matmul=True, vmem_limit_bytes=100 * 1024 * 1024),
    )(table_t)


def _make_gather(D2, B):
    info = plsc.get_sparse_core_info()
    NC, NS = info.num_cores, info.num_subcores
    NW = NC * NS
    b_per_w = B // NW
    mesh = plsc.VectorSubcoreMesh(core_axis_name="c", subcore_axis_name="s")

    @functools.partial(
        pl.kernel,
        mesh=mesh,
        out_type=jax.ShapeDtypeStruct((B, D2), jnp.float32),
        scratch_types=[
            pltpu.VMEM((b_per_w,), jnp.int32),
            pltpu.VMEM((b_per_w,), jnp.int32),
            pltpu.VMEM((b_per_w, D2), jnp.float32),
            pltpu.SemaphoreType.DMA,
        ],
    )
    def gather_k(idx_hbm, packed_hbm, out_hbm, idx_v, idx2_v, rows_v, sem):
        wid = lax.axis_index("s") * NC + lax.axis_index("c")
        base = wid * b_per_w
        pltpu.sync_copy(idx_hbm.at[pl.ds(base, b_per_w)], idx_v)
        for g in range(b_per_w // 16):
            sl = pl.ds(g * 16, 16)
            iv = idx_v[sl]
            idx2_v[sl] = lax.bitwise_or(
                lax.shift_left(lax.shift_right_logical(iv, _LG), _LG - 2),
                lax.bitwise_and(iv, (1 << (_LG - 2)) - 1),
            )
        pltpu.async_copy(packed_hbm.at[idx2_v], rows_v, sem).wait()
        pltpu.sync_copy(rows_v, out_hbm.at[pl.ds(base, b_per_w)])

    return gather_k


def _mm_body(x_ref, s_ref, wt_ref, b_ref, o_ref):
    xt = lax.transpose(x_ref[...], (1, 0))
    d = wt_ref.shape[0]
    s = s_ref[...]
    colhalf = lax.bitwise_and(s, 1) == 1
    lohalf = lax.bitwise_and(s, 2) == 2
    half = jnp.where(colhalf, xt[d:, :], xt[:d, :])
    bits = lax.bitcast_convert_type(half, jnp.int32)
    bits = jnp.where(
        lohalf,
        lax.shift_left(bits, 16),
        lax.bitwise_and(bits, jnp.int32(-65536)),
    )
    xsel = lax.bitcast_convert_type(bits, jnp.float32)
    o_ref[...] = (
        jnp.dot(wt_ref[...], xsel, preferred_element_type=jnp.float32)
        + b_ref[...]
    )


def _unpack_matmul_t(rows, subslot, wt, b2d):
    B, D2 = rows.shape
    D = D2 // 2
    blk = 2048
    return pl.pallas_call(
        _mm_body,
        grid=(B // blk,),
        in_specs=[
            pl.BlockSpec((blk, D2), lambda i: (i, 0)),
            pl.BlockSpec((1, blk), lambda i: (0, i)),
            pl.BlockSpec((D, D), lambda i: (0, 0)),
            pl.BlockSpec((D, 1), lambda i: (0, 0)),
        ],
        out_specs=pl.BlockSpec((D, blk), lambda i: (0, i)),
        out_shape=jax.ShapeDtypeStruct((D, B), jnp.float32),
    )(rows, subslot, wt, b2d)


def kernel(fixed_features, fixed_table, W, b):
    V, D = fixed_table.shape
    B = fixed_features.shape[0]
    packed = _pack(fixed_table.T)
    rows = _make_gather(2 * D, B)(fixed_features, packed)
    subslot = ((fixed_features >> (_LG - 2)) & 3).reshape(1, B)
    wtop_t = W.T[:, :D]
    out_t = _unpack_matmul_t(rows, subslot, wtop_t, b.reshape(D, 1))
    return out_t.T


# speedup vs baseline: 1.8550x; 1.0372x over previous
import functools

import jax
import jax.numpy as jnp
from jax import lax
from jax.experimental import pallas as pl
from jax.experimental.pallas import tpu as pltpu
from jax.experimental.pallas import tpu_sc as plsc


_LG = 15


def _pack_body(xt_ref, o_ref):
    xt = xt_ref[...]
    d = xt.shape[0]
    eye = (
        lax.broadcasted_iota(jnp.int32, (d, d), 0)
        == lax.broadcasted_iota(jnp.int32, (d, d), 1)
    ).astype(jnp.float32)
    t = lax.dot_general(
        xt, eye, (((0,), (0,)), ((), ())),
        preferred_element_type=jnp.float32,
    )
    ch = t.shape[0]
    a = lax.bitcast_convert_type(t[: ch // 2], jnp.int32)
    b_ = lax.bitcast_convert_type(t[ch // 2:], jnp.int32)
    hi = lax.bitwise_and(a + 0x8000, jnp.int32(-65536))
    lo = lax.shift_right_logical(b_ + 0x8000, 16)
    w1 = lax.bitcast_convert_type(lax.bitwise_or(hi, lo), jnp.float32)
    q = ch // 4
    o_ref[:, : w1.shape[1]] = w1[:q]
    o_ref[:, w1.shape[1]:] = w1[q:]


def _pack(table_t):
    D, V = table_t.shape
    ch = 1 << _LG
    grid = (V + ch - 1) // ch
    return pl.pallas_call(
        _pack_body,
        grid=(grid,),
        in_specs=[pl.BlockSpec((D, ch), lambda i: (0, i))],
        out_specs=pl.BlockSpec((ch // 4, 2 * D), lambda i: (i, 0)),
        out_shape=jax.ShapeDtypeStruct((grid * (ch // 4), 2 * D), jnp.float32),
        compiler_params=pltpu.CompilerParams(fuse_transposed_lhs_in_matmul=True),
    )(table_t)


def _make_gather(D2, B):
    info = plsc.get_sparse_core_info()
    NC, NS = info.num_cores, info.num_subcores
    NW = NC * NS
    b_per_w = B // NW
    mesh = plsc.VectorSubcoreMesh(core_axis_name="c", subcore_axis_name="s")

    @functools.partial(
        pl.kernel,
        mesh=mesh,
        out_type=jax.ShapeDtypeStruct((B, D2), jnp.float32),
        scratch_types=[
            pltpu.VMEM((b_per_w,), jnp.int32),
            pltpu.VMEM((b_per_w,), jnp.int32),
            pltpu.VMEM((b_per_w, D2), jnp.float32),
            pltpu.SemaphoreType.DMA,
        ],
    )
    def gather_k(idx_hbm, packed_hbm, out_hbm, idx_v, idx2_v, rows_v, sem):
        wid = lax.axis_index("s") * NC + lax.axis_index("c")
        base = wid * b_per_w
        pltpu.sync_copy(idx_hbm.at[pl.ds(base, b_per_w)], idx_v)
        for g in range(b_per_w // 16):
            sl = pl.ds(g * 16, 16)
            iv = idx_v[sl]
            idx2_v[sl] = lax.bitwise_or(
                lax.shift_left(lax.shift_right_logical(iv, _LG), _LG - 2),
                lax.bitwise_and(iv, (1 << (_LG - 2)) - 1),
            )
        pltpu.async_copy(packed_hbm.at[idx2_v], rows_v, sem).wait()
        pltpu.sync_copy(rows_v, out_hbm.at[pl.ds(base, b_per_w)])

    return gather_k


def _mm_body(x_ref, s_ref, wt_ref, b_ref, o_ref):
    xt = lax.transpose(x_ref[...], (1, 0))
    d = wt_ref.shape[0]
    s = s_ref[...]
    colhalf = lax.bitwise_and(s, 1) == 1
    lohalf = lax.bitwise_and(s, 2) == 2
    half = jnp.where(colhalf, xt[d:, :], xt[:d, :])
    bits = lax.bitcast_convert_type(half, jnp.int32)
    bits = jnp.where(
        lohalf,
        lax.shift_left(bits, 16),
        lax.bitwise_and(bits, jnp.int32(-65536)),
    )
    xsel = lax.bitcast_convert_type(bits, jnp.float32)
    o_ref[...] = (
        jnp.dot(wt_ref[...], xsel, preferred_element_type=jnp.float32)
        + b_ref[...]
    )


def _unpack_matmul_t(rows, subslot, wt, b2d):
    B, D2 = rows.shape
    D = D2 // 2
    blk = 2048
    return pl.pallas_call(
        _mm_body,
        grid=(B // blk,),
        in_specs=[
            pl.BlockSpec((blk, D2), lambda i: (i, 0)),
            pl.BlockSpec((1, blk), lambda i: (0, i)),
            pl.BlockSpec((D, D), lambda i: (0, 0)),
            pl.BlockSpec((D, 1), lambda i: (0, 0)),
        ],
        out_specs=pl.BlockSpec((D, blk), lambda i: (0, i)),
        out_shape=jax.ShapeDtypeStruct((D, B), jnp.float32),
    )(rows, subslot, wt, b2d)


def kernel(fixed_features, fixed_table, W, b):
    V, D = fixed_table.shape
    B = fixed_features.shape[0]
    packed = _pack(fixed_table.T)
    rows = _make_gather(2 * D, B)(fixed_features, packed)
    subslot = ((fixed_features >> (_LG - 2)) & 3).reshape(1, B)
    wtop_t = W.T[:, :D]
    out_t = _unpack_matmul_t(rows, subslot, wtop_t, b.reshape(D, 1))
    return out_t.T


# speedup vs baseline: 1.8697x; 1.0079x over previous
import functools

import jax
import jax.numpy as jnp
from jax import lax
from jax.experimental import pallas as pl
from jax.experimental.pallas import tpu as pltpu
from jax.experimental.pallas import tpu_sc as plsc


_LG = 15


def _pack_body(xt_ref, o_ref):
    xt = xt_ref[...]
    d = xt.shape[0]
    eye = (
        lax.broadcasted_iota(jnp.int32, (d, d), 0)
        == lax.broadcasted_iota(jnp.int32, (d, d), 1)
    ).astype(jnp.float32)
    t = lax.dot_general(
        xt, eye, (((0,), (0,)), ((), ())),
        preferred_element_type=jnp.float32,
    )
    ch = t.shape[0]
    a = lax.bitcast_convert_type(t[: ch // 2], jnp.int32)
    b_ = lax.bitcast_convert_type(t[ch // 2:], jnp.int32)
    hi = lax.bitwise_and(a + 0x8000, jnp.int32(-65536))
    lo = lax.shift_right_logical(b_ + 0x8000, 16)
    w1 = lax.bitcast_convert_type(lax.bitwise_or(hi, lo), jnp.float32)
    q = ch // 4
    o_ref[:, : w1.shape[1]] = w1[:q]
    o_ref[:, w1.shape[1]:] = w1[q:]


def _pack(table_t):
    D, V = table_t.shape
    ch = 1 << _LG
    grid = (V + ch - 1) // ch
    return pl.pallas_call(
        _pack_body,
        grid=(grid,),
        in_specs=[pl.BlockSpec((D, ch), lambda i: (0, i))],
        out_specs=pl.BlockSpec((ch // 4, 2 * D), lambda i: (i, 0)),
        out_shape=jax.ShapeDtypeStruct((grid * (ch // 4), 2 * D), jnp.float32),
        compiler_params=pltpu.CompilerParams(fuse_transposed_lhs_in_matmul=True),
    )(table_t)


def _make_gather(D2, B):
    info = plsc.get_sparse_core_info()
    NC, NS = info.num_cores, info.num_subcores
    NW = NC * NS
    b_per_w = B // NW
    mesh = plsc.VectorSubcoreMesh(core_axis_name="c", subcore_axis_name="s")

    @functools.partial(
        pl.kernel,
        mesh=mesh,
        out_type=jax.ShapeDtypeStruct((B, D2), jnp.float32),
        scratch_types=[
            pltpu.VMEM((b_per_w,), jnp.int32),
            pltpu.VMEM((b_per_w,), jnp.int32),
            pltpu.VMEM((b_per_w, D2), jnp.float32),
            pltpu.SemaphoreType.DMA,
        ],
    )
    def gather_k(idx_hbm, packed_hbm, out_hbm, idx_v, idx2_v, rows_v, sem):
        wid = lax.axis_index("s") * NC + lax.axis_index("c")
        base = wid * b_per_w
        pltpu.sync_copy(idx_hbm.at[pl.ds(base, b_per_w)], idx_v)
        for g in range(b_per_w // 16):
            sl = pl.ds(g * 16, 16)
            iv = idx_v[sl]
            idx2_v[sl] = lax.bitwise_or(
                lax.shift_left(lax.shift_right_logical(iv, _LG), _LG - 2),
                lax.bitwise_and(iv, (1 << (_LG - 2)) - 1),
            )
        pltpu.async_copy(packed_hbm.at[idx2_v], rows_v, sem).wait()
        pltpu.sync_copy(rows_v, out_hbm.at[pl.ds(base, b_per_w)])

    return gather_k


def _mm_body(x_ref, s_ref, wt_ref, b_ref, o_ref):
    xt = lax.transpose(x_ref[...], (1, 0))
    d = wt_ref.shape[0]
    s = s_ref[...]
    colhalf = lax.bitwise_and(s, 1) == 1
    lohalf = lax.bitwise_and(s, 2) == 2
    half = jnp.where(colhalf, xt[d:, :], xt[:d, :])
    bits = lax.bitcast_convert_type(half, jnp.int32)
    bits = jnp.where(
        lohalf,
        lax.shift_left(bits, 16),
        lax.bitwise_and(bits, jnp.int32(-65536)),
    )
    xsel = lax.bitcast_convert_type(bits, jnp.float32)
    o_ref[...] = (
        jnp.dot(wt_ref[...], xsel, preferred_element_type=jnp.float32)
        + b_ref[...]
    )


def _unpack_matmul_t(rows, subslot, wt, b2d):
    B, D2 = rows.shape
    D = D2 // 2
    blk = 4096
    return pl.pallas_call(
        _mm_body,
        grid=(B // blk,),
        in_specs=[
            pl.BlockSpec((blk, D2), lambda i: (i, 0)),
            pl.BlockSpec((1, blk), lambda i: (0, i)),
            pl.BlockSpec((D, D), lambda i: (0, 0)),
            pl.BlockSpec((D, 1), lambda i: (0, 0)),
        ],
        out_specs=pl.BlockSpec((D, blk), lambda i: (0, i)),
        out_shape=jax.ShapeDtypeStruct((D, B), jnp.float32),
    )(rows, subslot, wt, b2d)


def kernel(fixed_features, fixed_table, W, b):
    V, D = fixed_table.shape
    B = fixed_features.shape[0]
    packed = _pack(fixed_table.T)
    rows = _make_gather(2 * D, B)(fixed_features, packed)
    subslot = ((fixed_features >> (_LG - 2)) & 3).reshape(1, B)
    wtop_t = W.T[:, :D]
    out_t = _unpack_matmul_t(rows, subslot, wtop_t, b.reshape(D, 1))
    return out_t.T


# speedup vs baseline: 1.8839x; 1.0076x over previous
import functools

import jax
import jax.numpy as jnp
from jax import lax
from jax.experimental import pallas as pl
from jax.experimental.pallas import tpu as pltpu
from jax.experimental.pallas import tpu_sc as plsc


_LG = 15


def _pack_body(xt_ref, o_ref):
    xt = xt_ref[...]
    d = xt.shape[0]
    eye = (
        lax.broadcasted_iota(jnp.int32, (d, d), 0)
        == lax.broadcasted_iota(jnp.int32, (d, d), 1)
    ).astype(jnp.float32)
    t = lax.dot_general(
        xt, eye, (((0,), (0,)), ((), ())),
        preferred_element_type=jnp.float32,
    )
    ch = t.shape[0]
    a = lax.bitcast_convert_type(t[: ch // 2], jnp.int32)
    b_ = lax.bitcast_convert_type(t[ch // 2:], jnp.int32)
    hi = lax.bitwise_and(a + 0x8000, jnp.int32(-65536))
    lo = lax.shift_right_logical(b_ + 0x8000, 16)
    w1 = lax.bitcast_convert_type(lax.bitwise_or(hi, lo), jnp.float32)
    q = ch // 4
    o_ref[:, : w1.shape[1]] = w1[:q]
    o_ref[:, w1.shape[1]:] = w1[q:]


def _pack(table_t):
    D, V = table_t.shape
    ch = 1 << _LG
    grid = (V + ch - 1) // ch
    return pl.pallas_call(
        _pack_body,
        grid=(grid,),
        in_specs=[pl.BlockSpec((D, ch), lambda i: (0, i))],
        out_specs=pl.BlockSpec((ch // 4, 2 * D), lambda i: (i, 0)),
        out_shape=jax.ShapeDtypeStruct((grid * (ch // 4), 2 * D), jnp.float32),
        compiler_params=pltpu.CompilerParams(fuse_transposed_lhs_in_matmul=True),
    )(table_t)


def _make_gather(D2, B):
    info = plsc.get_sparse_core_info()
    NC, NS = info.num_cores, info.num_subcores
    NW = NC * NS
    b_per_w = B // NW
    mesh = plsc.VectorSubcoreMesh(core_axis_name="c", subcore_axis_name="s")

    @functools.partial(
        pl.kernel,
        mesh=mesh,
        out_type=jax.ShapeDtypeStruct((B, D2), jnp.float32),
        scratch_types=[
            pltpu.VMEM((b_per_w,), jnp.int32),
            pltpu.VMEM((b_per_w,), jnp.int32),
            pltpu.VMEM((b_per_w, D2), jnp.float32),
            pltpu.SemaphoreType.DMA,
        ],
    )
    def gather_k(idx_hbm, packed_hbm, out_hbm, idx_v, idx2_v, rows_v, sem):
        wid = lax.axis_index("s") * NC + lax.axis_index("c")
        base = wid * b_per_w
        pltpu.sync_copy(idx_hbm.at[pl.ds(base, b_per_w)], idx_v)
        for g in range(b_per_w // 16):
            sl = pl.ds(g * 16, 16)
            iv = idx_v[sl]
            idx2_v[sl] = lax.bitwise_or(
                lax.shift_left(lax.shift_right_logical(iv, _LG), _LG - 2),
                lax.bitwise_and(iv, (1 << (_LG - 2)) - 1),
            )
        pltpu.async_copy(packed_hbm.at[idx2_v], rows_v, sem).wait()
        pltpu.sync_copy(rows_v, out_hbm.at[pl.ds(base, b_per_w)])

    return gather_k


def _mm_body(x_ref, s_ref, wt_ref, b_ref, o_ref):
    xt = lax.transpose(x_ref[...], (1, 0))
    d = wt_ref.shape[0]
    s = s_ref[...]
    colhalf = lax.bitwise_and(s, 1) == 1
    lohalf = lax.bitwise_and(s, 2) == 2
    half = jnp.where(colhalf, xt[d:, :], xt[:d, :])
    bits = lax.bitcast_convert_type(half, jnp.int32)
    bits = jnp.where(
        lohalf,
        lax.shift_left(bits, 16),
        lax.bitwise_and(bits, jnp.int32(-65536)),
    )
    xsel = lax.bitcast_convert_type(bits, jnp.float32)
    o_ref[...] = (
        jnp.dot(wt_ref[...], xsel, preferred_element_type=jnp.float32)
        + b_ref[...]
    )


def _unpack_matmul_t(rows, subslot, wt, b2d):
    B, D2 = rows.shape
    D = D2 // 2
    blk = 8192
    return pl.pallas_call(
        _mm_body,
        grid=(B // blk,),
        in_specs=[
            pl.BlockSpec((blk, D2), lambda i: (i, 0)),
            pl.BlockSpec((1, blk), lambda i: (0, i)),
            pl.BlockSpec((D, D), lambda i: (0, 0)),
            pl.BlockSpec((D, 1), lambda i: (0, 0)),
        ],
        out_specs=pl.BlockSpec((D, blk), lambda i: (0, i)),
        out_shape=jax.ShapeDtypeStruct((D, B), jnp.float32),
    )(rows, subslot, wt, b2d)


def kernel(fixed_features, fixed_table, W, b):
    V, D = fixed_table.shape
    B = fixed_features.shape[0]
    packed = _pack(fixed_table.T)
    rows = _make_gather(2 * D, B)(fixed_features, packed)
    subslot = ((fixed_features >> (_LG - 2)) & 3).reshape(1, B)
    wtop_t = W.T[:, :D]
    out_t = _unpack_matmul_t(rows, subslot, wtop_t, b.reshape(D, 1))
    return out_t.T
